# Initial kernel scaffold; baseline (speedup 1.0000x reference)
import jax
import jax.numpy as jnp
from jax.experimental import pallas as pl


def kernel(z, emb_w):
    raise NotImplementedError("write your pallas kernel here")


# speedup vs baseline: 1.1120x; 1.1120x over previous
import functools

import jax
import jax.numpy as jnp
from jax.experimental import pallas as pl
from jax.experimental.pallas import tpu as pltpu

N_E = 8192
E_DIM = 32
BETA = 0.25
TM = 256


def _vq_tile_kernel(z_ref, a_ref, b_ref, w_ref,
                    d_ref, oh_ref, idx_ref, zq_ref, cnt_ref, ls_ref):
    i = pl.program_id(0)
    z = z_ref[...]
    w = w_ref[...]
    c = jax.lax.dot_general(z, w, (((1,), (1,)), ((), ())),
                            preferred_element_type=jnp.float32)
    d = (a_ref[...] + b_ref[...]) - 2.0 * c
    d_ref[...] = d
    minv = jnp.min(d, axis=1, keepdims=True)
    iota = jax.lax.broadcasted_iota(jnp.int32, d.shape, 1)
    idx = jnp.min(jnp.where(d == minv, iota, N_E), axis=1)
    idx_ref[...] = idx[:, None]
    oh = (iota == idx[:, None]).astype(jnp.float32)
    oh_ref[...] = oh
    zq = jax.lax.dot_general(oh, w, (((1,), (0,)), ((), ())),
                             preferred_element_type=jnp.float32,
                             precision=jax.lax.Precision.HIGHEST)
    zq_ref[...] = zq
    diff = zq - z

    @pl.when(i == 0)
    def _init():
        cnt_ref[...] = jnp.zeros_like(cnt_ref)
        ls_ref[...] = jnp.zeros_like(ls_ref)

    cnt_ref[...] += jnp.sum(oh, axis=0, keepdims=True)
    ls_ref[...] += jnp.sum(diff * diff, keepdims=True)


@jax.jit
def kernel(z, emb_w):
    B, C, H, W = z.shape
    M = B * H * W
    z_perm = jnp.transpose(z, (0, 2, 3, 1))
    z_flat = z_perm.reshape(-1, E_DIM)
    a = jnp.sum(z_flat ** 2, axis=1, keepdims=True)
    b = jnp.sum(emb_w ** 2, axis=1)[None, :]

    grid = (M // TM,)
    d, oh, idx, zq_flat, cnt, ls = pl.pallas_call(
        _vq_tile_kernel,
        grid=grid,
        in_specs=[
            pl.BlockSpec((TM, E_DIM), lambda i: (i, 0)),
            pl.BlockSpec((TM, 1), lambda i: (i, 0)),
            pl.BlockSpec((1, N_E), lambda i: (0, 0)),
            pl.BlockSpec((N_E, E_DIM), lambda i: (0, 0)),
        ],
        out_specs=[
            pl.BlockSpec((TM, N_E), lambda i: (i, 0)),
            pl.BlockSpec((TM, N_E), lambda i: (i, 0)),
            pl.BlockSpec((TM, 1), lambda i: (i, 0)),
            pl.BlockSpec((TM, E_DIM), lambda i: (i, 0)),
            pl.BlockSpec((1, N_E), lambda i: (0, 0)),
            pl.BlockSpec((1, 1), lambda i: (0, 0)),
        ],
        out_shape=[
            jax.ShapeDtypeStruct((M, N_E), jnp.float32),
            jax.ShapeDtypeStruct((M, N_E), jnp.float32),
            jax.ShapeDtypeStruct((M, 1), jnp.int32),
            jax.ShapeDtypeStruct((M, E_DIM), jnp.float32),
            jax.ShapeDtypeStruct((1, N_E), jnp.float32),
            jax.ShapeDtypeStruct((1, 1), jnp.float32),
        ],
        compiler_params=pltpu.CompilerParams(
            dimension_semantics=("arbitrary",)),
    )(z_flat, a, b, emb_w)

    loss = (ls[0, 0] / (M * E_DIM)) * (1.0 + BETA)
    e_mean = cnt[0] / M
    perplexity = jnp.exp(-jnp.sum(e_mean * jnp.log(e_mean + 1e-10)))
    z_q = z_flat + (zq_flat - z_flat)
    z_q_out = jnp.transpose(z_q.reshape(B, H, W, C), (0, 3, 1, 2))
    return (z_q_out, loss, perplexity, oh, idx, d)
